# baseline (speedup 1.0000x reference)
import functools
import jax
import jax.numpy as jnp
from jax import lax
from jax.experimental import pallas as pl
from jax.experimental.pallas import tpu as pltpu
from jax.experimental.pallas import tpu_sc as plsc

B = 1024
L = 200
D = 64
V = 100000
HOPS = 3
NT = 4

NC = 2
NS = 16
NW = NC * NS
NB = 4
CB = B // NB
RPW = CB // NW
LP = 208
NCH = LP // 16
IC = 2
ICL = LP // IC
NEG = -1e30


def _splat(x):
    return jnp.full((16,), x, jnp.int32)


def _bcast_lane(vec, lane):
    dn = lax.GatherDimensionNumbers(
        offset_dims=(), collapsed_slice_dims=(0,), start_index_map=(0,))
    idx = jnp.full((16, 1), lane, jnp.int32)
    return lax.gather(vec, idx, dn, slice_sizes=(1,),
                      mode=lax.GatherScatterMode.PROMISE_IN_BOUNDS)


def _sc_attn_body(ids_hbm, hid_hbm, t0, t1, t2, t3, u_hbm,
                  gbuf, idxb, ubuf, lbuf, ebuf, sem0, sem1):
    tables = [t0, t1, t2, t3]
    sems = [sem0, sem1]
    iota16 = lax.iota(jnp.int32, 16)
    lane0 = iota16 == 0
    wid = lax.axis_index("s") * NC + lax.axis_index("c")
    row0 = wid * RPW

    def fire(row, slot):
        pltpu.sync_copy(ids_hbm.at[row], idxb.at[slot])
        for t in range(NT):
            for c in range(IC):
                pltpu.async_copy(
                    tables[t].at[idxb.at[slot, c]],
                    gbuf.at[slot, t, pl.ds(c * ICL, ICL)],
                    sems[slot])

    def drain(row, slot):
        for t in range(NT):
            for c in range(IC):
                pltpu.make_async_copy(
                    tables[t].at[idxb.at[slot, c]],
                    gbuf.at[slot, t, pl.ds(c * ICL, ICL)],
                    sems[slot]).wait()

    def compute(row, slot):
        pltpu.sync_copy(hid_hbm.at[row], ubuf)

        def hop_body(h, _):
            up = [ubuf[pl.ds(16 * k, 16)] for k in range(4)]
            zero = jnp.zeros((16,), jnp.float32)

            def logit_c(c, _c):
                lvec = c * 16 + iota16
                lacc = zero
                for j in range(16):
                    lrow = c * 16 + j
                    p01 = (up[0] * gbuf[slot, h, lrow, pl.ds(0, 16)]
                           + up[1] * gbuf[slot, h, lrow, pl.ds(16, 16)])
                    p23 = (up[2] * gbuf[slot, h, lrow, pl.ds(32, 16)]
                           + up[3] * gbuf[slot, h, lrow, pl.ds(48, 16)])
                    cs = plsc.cumsum(p01 + p23)
                    sv = _bcast_lane(cs, 15)
                    lacc = jnp.where(iota16 == j, sv, lacc)
                lacc = jnp.where(lvec < L, lacc, NEG)
                lbuf[pl.ds(c * 16, 16)] = lacc
                return 0

            lax.fori_loop(0, NCH, logit_c, 0)

            mv = lbuf[pl.ds(0, 16)]
            for c in range(1, NCH):
                mv = jnp.maximum(mv, lbuf[pl.ds(c * 16, 16)])
            m = jnp.max(mv)
            sacc = jnp.zeros((16,), jnp.float32)
            for c in range(NCH):
                e = jnp.exp(lbuf[pl.ds(c * 16, 16)] - m)
                ebuf[pl.ds(c * 16, 16)] = e
                sacc = sacc + e
            sv = jnp.zeros((16,), jnp.float32) + jnp.sum(sacc)
            inv = jnp.ones((16,), jnp.float32) / sv

            def o_c(c, accs):
                e_c = ebuf[pl.ds(c * 16, 16)]
                new = list(accs)
                for j in range(16):
                    eb = _bcast_lane(e_c, j)
                    lrow = c * 16 + j
                    for k in range(4):
                        g = gbuf[slot, h + 1, lrow, pl.ds(16 * k, 16)]
                        a = k * 2 + (j % 2)
                        new[a] = new[a] + eb * g
                return tuple(new)

            accs = lax.fori_loop(0, NCH, o_c, (zero,) * 8)
            for k in range(4):
                ok = accs[k * 2] + accs[k * 2 + 1]
                ubuf[pl.ds(16 * k, 16)] = up[k] + inv * ok
            return 0

        lax.fori_loop(0, HOPS, hop_body, 0)
        pltpu.sync_copy(ubuf, u_hbm.at[row])

    fire(row0, 0)

    def pair_body(i, _):
        r = row0 + 2 * i
        for s in (0, 1):
            row = r + s
            nxt = row + 1

            @pl.when(nxt < row0 + RPW)
            def _():
                fire(nxt, 1 - s)

            drain(row, s)
            compute(row, s)
        return 0

    lax.fori_loop(0, RPW // 2, pair_body, 0)


def _sc_attention(ids3, hidden, t0, t1, t2, t3):
    mesh = plsc.VectorSubcoreMesh(core_axis_name="c", subcore_axis_name="s")
    return pl.kernel(
        _sc_attn_body,
        out_type=jax.ShapeDtypeStruct((CB, D), jnp.float32),
        mesh=mesh,
        scratch_types=[
            pltpu.VMEM((2, NT, LP, D), jnp.float32),
            pltpu.VMEM((2, IC, ICL), jnp.int32),
            pltpu.VMEM((D,), jnp.float32),
            pltpu.VMEM((LP,), jnp.float32),
            pltpu.VMEM((LP,), jnp.float32),
            pltpu.SemaphoreType.DMA,
            pltpu.SemaphoreType.DMA,
        ],
        compiler_params=pltpu.CompilerParams(
            use_tc_tiling_on_sc=False, needs_layout_passes=False),
    )(ids3, hidden, t0, t1, t2, t3)


VB = 2048


def _classifier_body(u_ref, w_ref, b_ref, o_ref):
    acc = lax.dot_general(
        u_ref[...], w_ref[...],
        dimension_numbers=(((1,), (1,)), ((), ())),
        preferred_element_type=jnp.float32,
    )
    o_ref[...] = jax.nn.sigmoid(acc + b_ref[...])


def _classifier_next_body(prev_ref, u_ref, w_ref, b_ref, o_ref):
    acc = lax.dot_general(
        u_ref[...], w_ref[...],
        dimension_numbers=(((1,), (1,)), ((), ())),
        preferred_element_type=jnp.float32,
    )
    o_ref[...] = jax.nn.sigmoid(acc + b_ref[...])


def _classifier_chunk(u, W, b2, ci, prev):
    nvb = pl.cdiv(V, VB)
    uspec = pl.BlockSpec((CB, D), lambda j: (0, 0))
    wspec = pl.BlockSpec((VB, D), lambda j: (j, 0))
    bspec = pl.BlockSpec((1, VB), lambda j: (0, j))
    ospec = pl.BlockSpec((CB, VB), lambda j, ci=ci: (ci, j))
    oshape = jax.ShapeDtypeStruct((B, V), jnp.float32)
    if prev is None:
        return pl.pallas_call(
            _classifier_body,
            grid=(nvb,),
            in_specs=[uspec, wspec, bspec],
            out_specs=ospec,
            out_shape=oshape,
        )(u, W, b2)
    return pl.pallas_call(
        _classifier_next_body,
        grid=(nvb,),
        in_specs=[pl.BlockSpec(memory_space=pl.ANY), uspec, wspec, bspec],
        out_specs=ospec,
        out_shape=oshape,
        input_output_aliases={0: 0},
    )(prev, u, W, b2)


def kernel(input_ids, hidden_states, C0, C1, C2, C3, W, b):
    ids = input_ids.astype(jnp.int32)
    ids_pad = jnp.pad(ids, ((0, 0), (0, LP - L))).reshape(B, IC, ICL)
    b2 = b.reshape(1, V)
    us = []
    for ci in range(NB):
        sl = slice(ci * CB, (ci + 1) * CB)
        us.append(_sc_attention(ids_pad[sl], hidden_states[sl],
                                C0, C1, C2, C3))
    out = None
    for ci in range(NB):
        out = _classifier_chunk(us[ci], W, b2, ci, out)
    return out


# speedup vs baseline: 1.0243x; 1.0243x over previous
import functools
import jax
import jax.numpy as jnp
from jax import lax
from jax.experimental import pallas as pl
from jax.experimental.pallas import tpu as pltpu
from jax.experimental.pallas import tpu_sc as plsc

B = 1024
L = 200
D = 64
V = 100000
HOPS = 3
NT = 4

NC = 2
NS = 16
NW = NC * NS
NB = 4
CB = B // NB
RPW = CB // NW
LP = 208
NCH = LP // 16
IC = 2
ICL = LP // IC
NEG = -1e30


def _splat(x):
    return jnp.full((16,), x, jnp.int32)


def _bcast_lane(vec, lane):
    dn = lax.GatherDimensionNumbers(
        offset_dims=(), collapsed_slice_dims=(0,), start_index_map=(0,))
    idx = jnp.full((16, 1), lane, jnp.int32)
    return lax.gather(vec, idx, dn, slice_sizes=(1,),
                      mode=lax.GatherScatterMode.PROMISE_IN_BOUNDS)


def _sc_attn_body(ci, ids_hbm, hid_hbm, t0, t1, t2, t3, u_hbm,
                  gbuf, idxb, ubuf, lbuf, ebuf, sem0, sem1):
    tables = [t0, t1, t2, t3]
    sems = [sem0, sem1]
    iota16 = lax.iota(jnp.int32, 16)
    wid = lax.axis_index("s") * NC + lax.axis_index("c")
    row0 = ci * CB + wid * RPW

    for slot in range(2):
        idxb[slot, pl.ds(192, 16)] = jnp.zeros((16,), jnp.int32)

    def fire(row, slot):
        pltpu.sync_copy(ids_hbm.at[row], idxb.at[slot, pl.ds(0, L)])
        for t in range(NT):
            for c in range(IC):
                pltpu.async_copy(
                    tables[t].at[idxb.at[slot, pl.ds(c * ICL, ICL)]],
                    gbuf.at[slot, t, pl.ds(c * ICL, ICL)],
                    sems[slot])

    def drain(row, slot):
        for t in range(NT):
            for c in range(IC):
                pltpu.make_async_copy(
                    tables[t].at[idxb.at[slot, pl.ds(c * ICL, ICL)]],
                    gbuf.at[slot, t, pl.ds(c * ICL, ICL)],
                    sems[slot]).wait()

    def compute(row, slot):
        pltpu.sync_copy(hid_hbm.at[row], ubuf)

        def hop_body(h, _):
            up = [ubuf[pl.ds(16 * k, 16)] for k in range(4)]
            zero = jnp.zeros((16,), jnp.float32)

            def logit_c(c, _c):
                lvec = c * 16 + iota16
                lacc = zero
                for j in range(16):
                    lrow = c * 16 + j
                    p01 = (up[0] * gbuf[slot, h, lrow, pl.ds(0, 16)]
                           + up[1] * gbuf[slot, h, lrow, pl.ds(16, 16)])
                    p23 = (up[2] * gbuf[slot, h, lrow, pl.ds(32, 16)]
                           + up[3] * gbuf[slot, h, lrow, pl.ds(48, 16)])
                    cs = plsc.cumsum(p01 + p23)
                    sv = _bcast_lane(cs, 15)
                    lacc = jnp.where(iota16 == j, sv, lacc)
                lacc = jnp.where(lvec < L, lacc, NEG)
                lbuf[pl.ds(c * 16, 16)] = lacc
                return 0

            lax.fori_loop(0, NCH, logit_c, 0)

            mv = lbuf[pl.ds(0, 16)]
            for c in range(1, NCH):
                mv = jnp.maximum(mv, lbuf[pl.ds(c * 16, 16)])
            m = jnp.max(mv)
            sacc = jnp.zeros((16,), jnp.float32)
            for c in range(NCH):
                e = jnp.exp(lbuf[pl.ds(c * 16, 16)] - m)
                ebuf[pl.ds(c * 16, 16)] = e
                sacc = sacc + e
            sv = jnp.zeros((16,), jnp.float32) + jnp.sum(sacc)
            inv = jnp.ones((16,), jnp.float32) / sv

            def o_c(c, accs):
                e_c = ebuf[pl.ds(c * 16, 16)]
                new = list(accs)
                for j in range(16):
                    eb = _bcast_lane(e_c, j)
                    lrow = c * 16 + j
                    for k in range(4):
                        g = gbuf[slot, h + 1, lrow, pl.ds(16 * k, 16)]
                        a = k * 2 + (j % 2)
                        new[a] = new[a] + eb * g
                return tuple(new)

            accs = lax.fori_loop(0, NCH, o_c, (zero,) * 8)
            for k in range(4):
                ok = accs[k * 2] + accs[k * 2 + 1]
                ubuf[pl.ds(16 * k, 16)] = up[k] + inv * ok
            return 0

        lax.fori_loop(0, HOPS, hop_body, 0)
        pltpu.sync_copy(ubuf, u_hbm.at[row - ci * CB])

    fire(row0, 0)

    def pair_body(i, _):
        r = row0 + 2 * i
        for s in (0, 1):
            row = r + s
            nxt = row + 1

            @pl.when(nxt < row0 + RPW)
            def _():
                fire(nxt, 1 - s)

            drain(row, s)
            compute(row, s)
        return 0

    lax.fori_loop(0, RPW // 2, pair_body, 0)


def _sc_attention(ci, ids, hidden, t0, t1, t2, t3):
    mesh = plsc.VectorSubcoreMesh(core_axis_name="c", subcore_axis_name="s")
    return pl.kernel(
        functools.partial(_sc_attn_body, ci),
        out_type=jax.ShapeDtypeStruct((CB, D), jnp.float32),
        mesh=mesh,
        scratch_types=[
            pltpu.VMEM((2, NT, LP, D), jnp.float32),
            pltpu.VMEM((2, LP), jnp.int32),
            pltpu.VMEM((D,), jnp.float32),
            pltpu.VMEM((LP,), jnp.float32),
            pltpu.VMEM((LP,), jnp.float32),
            pltpu.SemaphoreType.DMA,
            pltpu.SemaphoreType.DMA,
        ],
        compiler_params=pltpu.CompilerParams(
            use_tc_tiling_on_sc=False, needs_layout_passes=False),
    )(ids, hidden, t0, t1, t2, t3)


VB = 2048


def _classifier(us, W, b2):
    nvb = pl.cdiv(V, VB)

    def body(u0, u1, u2, u3, w_ref, b_ref, o_ref):
        i = pl.program_id(1)
        uref = [u0, u1, u2, u3]
        for k in range(NB):
            @pl.when(i == k)
            def _():
                o_ref[...] = jax.nn.sigmoid(
                    lax.dot_general(
                        uref[k][...], w_ref[...],
                        dimension_numbers=(((1,), (1,)), ((), ())),
                        preferred_element_type=jnp.float32,
                    ) + b_ref[...])

    uspec = pl.BlockSpec((CB, D), lambda j, i: (0, 0))
    return pl.pallas_call(
        body,
        grid=(nvb, NB),
        in_specs=[uspec] * NB + [
            pl.BlockSpec((VB, D), lambda j, i: (j, 0)),
            pl.BlockSpec((1, VB), lambda j, i: (0, j)),
        ],
        out_specs=pl.BlockSpec((CB, VB), lambda j, i: (i, j)),
        out_shape=jax.ShapeDtypeStruct((B, V), jnp.float32),
    )(*us, W, b2)


def kernel(input_ids, hidden_states, C0, C1, C2, C3, W, b):
    ids = input_ids.astype(jnp.int32)
    b2 = b.reshape(1, V)
    us = [_sc_attention(ci, ids, hidden_states, C0, C1, C2, C3)
          for ci in range(NB)]
    return _classifier(us, W, b2)


# speedup vs baseline: 1.0462x; 1.0214x over previous
import functools
import jax
import jax.numpy as jnp
from jax import lax
from jax.experimental import pallas as pl
from jax.experimental.pallas import tpu as pltpu
from jax.experimental.pallas import tpu_sc as plsc

B = 1024
L = 200
D = 64
V = 100000
HOPS = 3
NT = 4

NC = 2
NS = 16
NW = NC * NS
NB = 4
CB = B // NB
RPW = CB // NW
LP = 208
NCH = LP // 16
IC = 2
ICL = LP // IC
NEG = -1e30


def _splat(x):
    return jnp.full((16,), x, jnp.int32)


def _bcast_lane(vec, lane):
    dn = lax.GatherDimensionNumbers(
        offset_dims=(), collapsed_slice_dims=(0,), start_index_map=(0,))
    idx = jnp.full((16, 1), lane, jnp.int32)
    return lax.gather(vec, idx, dn, slice_sizes=(1,),
                      mode=lax.GatherScatterMode.PROMISE_IN_BOUNDS)


def _sc_attn_body(ci, ids_hbm, hid_hbm, t0, t1, t2, t3, u_hbm,
                  gbuf, idxb, ubuf, lbuf, ebuf, sem0, sem1):
    tables = [t0, t1, t2, t3]
    sems = [sem0, sem1]
    iota16 = lax.iota(jnp.int32, 16)
    wid = lax.axis_index("s") * NC + lax.axis_index("c")
    row0 = ci * CB + wid * RPW

    for slot in range(2):
        idxb[slot, pl.ds(192, 16)] = jnp.zeros((16,), jnp.int32)

    def fire(row, slot):
        pltpu.sync_copy(ids_hbm.at[row], idxb.at[slot, pl.ds(0, L)])
        for t in range(NT):
            for c in range(IC):
                pltpu.async_copy(
                    tables[t].at[idxb.at[slot, pl.ds(c * ICL, ICL)]],
                    gbuf.at[slot, t, pl.ds(c * ICL, ICL)],
                    sems[slot])

    def drain(row, slot):
        for t in range(NT):
            for c in range(IC):
                pltpu.make_async_copy(
                    tables[t].at[idxb.at[slot, pl.ds(c * ICL, ICL)]],
                    gbuf.at[slot, t, pl.ds(c * ICL, ICL)],
                    sems[slot]).wait()

    def compute(row, slot):
        pltpu.sync_copy(hid_hbm.at[row], ubuf)

        def hop_body(h, _):
            up = [ubuf[pl.ds(16 * k, 16)] for k in range(4)]
            zero = jnp.zeros((16,), jnp.float32)

            def logit_c(c, _c):
                lvec = c * 16 + iota16
                lacc = zero
                for j in range(16):
                    lrow = c * 16 + j
                    p01 = (up[0] * gbuf[slot, h, lrow, pl.ds(0, 16)]
                           + up[1] * gbuf[slot, h, lrow, pl.ds(16, 16)])
                    p23 = (up[2] * gbuf[slot, h, lrow, pl.ds(32, 16)]
                           + up[3] * gbuf[slot, h, lrow, pl.ds(48, 16)])
                    cs = plsc.cumsum(p01 + p23)
                    sv = _bcast_lane(cs, 15)
                    lacc = jnp.where(iota16 == j, sv, lacc)
                lacc = jnp.where(lvec < L, lacc, NEG)
                lbuf[pl.ds(c * 16, 16)] = lacc
                return 0

            lax.fori_loop(0, NCH, logit_c, 0)

            mv = lbuf[pl.ds(0, 16)]
            for c in range(1, NCH):
                mv = jnp.maximum(mv, lbuf[pl.ds(c * 16, 16)])
            m = jnp.max(mv)
            sacc = jnp.zeros((16,), jnp.float32)
            for c in range(NCH):
                e = jnp.exp(lbuf[pl.ds(c * 16, 16)] - m)
                ebuf[pl.ds(c * 16, 16)] = e
                sacc = sacc + e
            sv = jnp.zeros((16,), jnp.float32) + jnp.sum(sacc)
            inv = jnp.ones((16,), jnp.float32) / sv

            def o_c(c, accs):
                e_c = ebuf[pl.ds(c * 16, 16)]
                new = list(accs)
                for j in range(16):
                    eb = _bcast_lane(e_c, j)
                    lrow = c * 16 + j
                    for k in range(4):
                        g = gbuf[slot, h + 1, lrow, pl.ds(16 * k, 16)]
                        a = k * 2 + (j % 2)
                        new[a] = new[a] + eb * g
                return tuple(new)

            accs = lax.fori_loop(0, NCH, o_c, (zero,) * 8)
            for k in range(4):
                ok = accs[k * 2] + accs[k * 2 + 1]
                ubuf[pl.ds(16 * k, 16)] = up[k] + inv * ok
            return 0

        lax.fori_loop(0, HOPS, hop_body, 0)
        pltpu.sync_copy(ubuf, u_hbm.at[row - ci * CB])

    fire(row0, 0)

    def pair_body(i, _):
        r = row0 + 2 * i
        for s in (0, 1):
            row = r + s
            nxt = row + 1

            @pl.when(nxt < row0 + RPW)
            def _():
                fire(nxt, 1 - s)

            drain(row, s)
            compute(row, s)
        return 0

    lax.fori_loop(0, RPW // 2, pair_body, 0)


def _sc_attention(ci, ids, hidden, t0, t1, t2, t3):
    mesh = plsc.VectorSubcoreMesh(core_axis_name="c", subcore_axis_name="s")
    return pl.kernel(
        functools.partial(_sc_attn_body, ci),
        out_type=jax.ShapeDtypeStruct((CB, D), jnp.float32),
        mesh=mesh,
        scratch_types=[
            pltpu.VMEM((2, NT, LP, D), jnp.float32),
            pltpu.VMEM((2, LP), jnp.int32),
            pltpu.VMEM((D,), jnp.float32),
            pltpu.VMEM((LP,), jnp.float32),
            pltpu.VMEM((LP,), jnp.float32),
            pltpu.SemaphoreType.DMA,
            pltpu.SemaphoreType.DMA,
        ],
        compiler_params=pltpu.CompilerParams(
            use_tc_tiling_on_sc=False, needs_layout_passes=False),
    )(ids, hidden, t0, t1, t2, t3)


VB = 512


def _classifier(us, W, b2):
    nvb = pl.cdiv(V, VB)

    def body(u0, u1, u2, u3, w_ref, b_ref, o_ref):
        uref = [u0, u1, u2, u3]
        for k in range(NB):
            o_ref[pl.ds(k * CB, CB), :] = jax.nn.sigmoid(
                lax.dot_general(
                    uref[k][...], w_ref[...],
                    dimension_numbers=(((1,), (1,)), ((), ())),
                    preferred_element_type=jnp.float32,
                ) + b_ref[...])

    uspec = pl.BlockSpec((CB, D), lambda j: (0, 0))
    return pl.pallas_call(
        body,
        grid=(nvb,),
        in_specs=[uspec] * NB + [
            pl.BlockSpec((VB, D), lambda j: (j, 0)),
            pl.BlockSpec((1, VB), lambda j: (0, j)),
        ],
        out_specs=pl.BlockSpec((B, VB), lambda j: (0, j)),
        out_shape=jax.ShapeDtypeStruct((B, V), jnp.float32),
    )(*us, W, b2)


def kernel(input_ids, hidden_states, C0, C1, C2, C3, W, b):
    ids = input_ids.astype(jnp.int32)
    b2 = b.reshape(1, V)
    us = [_sc_attention(ci, ids, hidden_states, C0, C1, C2, C3)
          for ci in range(NB)]
    return _classifier(us, W, b2)


# speedup vs baseline: 1.4707x; 1.4058x over previous
import functools
import jax
import jax.numpy as jnp
from jax import lax
from jax.experimental import pallas as pl
from jax.experimental.pallas import tpu as pltpu
from jax.experimental.pallas import tpu_sc as plsc

B = 1024
L = 200
D = 64
V = 100000
HOPS = 3
NT = 4

NC = 2
NS = 16
NW = NC * NS
NB = 4
CB = B // NB
RPW = CB // NW
LP = 208
NCH = LP // 16
IC = 2
ICL = LP // IC
NEG = -1e30


def _splat(x):
    return jnp.full((16,), x, jnp.int32)


def _bcast_lane(vec, lane):
    dn = lax.GatherDimensionNumbers(
        offset_dims=(), collapsed_slice_dims=(0,), start_index_map=(0,))
    idx = jnp.full((16, 1), lane, jnp.int32)
    return lax.gather(vec, idx, dn, slice_sizes=(1,),
                      mode=lax.GatherScatterMode.PROMISE_IN_BOUNDS)


def _sc_attn_body(ci, ids_hbm, hid_hbm, t0, t1, t2, t3, u_hbm,
                  gbuf, idxb, ubuf, lbuf, ebuf, sem0, sem1):
    tables = [t0, t1, t2, t3]
    sems = [sem0, sem1]
    iota16 = lax.iota(jnp.int32, 16)
    wid = lax.axis_index("s") * NC + lax.axis_index("c")
    row0 = ci * CB + wid * RPW

    for slot in range(2):
        idxb[slot, pl.ds(192, 16)] = jnp.zeros((16,), jnp.int32)

    def fire(row, slot):
        pltpu.sync_copy(ids_hbm.at[row], idxb.at[slot, pl.ds(0, L)])
        for t in range(NT):
            for c in range(IC):
                pltpu.async_copy(
                    tables[t].at[idxb.at[slot, pl.ds(c * ICL, ICL)]],
                    gbuf.at[slot, t, pl.ds(c * ICL, ICL)],
                    sems[slot])

    def drain(row, slot):
        for t in range(NT):
            for c in range(IC):
                pltpu.make_async_copy(
                    tables[t].at[idxb.at[slot, pl.ds(c * ICL, ICL)]],
                    gbuf.at[slot, t, pl.ds(c * ICL, ICL)],
                    sems[slot]).wait()

    def compute(row, slot):
        pltpu.sync_copy(hid_hbm.at[row], ubuf)

        def hop_body(h, _):
            up = [ubuf[pl.ds(16 * k, 16)] for k in range(4)]
            zero = jnp.zeros((16,), jnp.float32)

            def logit_c(c, _c):
                lvec = c * 16 + iota16
                lacc = zero
                for j in range(16):
                    lrow = c * 16 + j
                    p01 = (up[0] * gbuf[slot, h, lrow, pl.ds(0, 16)]
                           + up[1] * gbuf[slot, h, lrow, pl.ds(16, 16)])
                    p23 = (up[2] * gbuf[slot, h, lrow, pl.ds(32, 16)]
                           + up[3] * gbuf[slot, h, lrow, pl.ds(48, 16)])
                    cs = plsc.cumsum(p01 + p23)
                    sv = _bcast_lane(cs, 15)
                    lacc = jnp.where(iota16 == j, sv, lacc)
                lacc = jnp.where(lvec < L, lacc, NEG)
                lbuf[pl.ds(c * 16, 16)] = lacc
                return 0

            lax.fori_loop(0, NCH, logit_c, 0)

            mv = lbuf[pl.ds(0, 16)]
            for c in range(1, NCH):
                mv = jnp.maximum(mv, lbuf[pl.ds(c * 16, 16)])
            m = jnp.max(mv)
            sacc = jnp.zeros((16,), jnp.float32)
            for c in range(NCH):
                e = jnp.exp(lbuf[pl.ds(c * 16, 16)] - m)
                ebuf[pl.ds(c * 16, 16)] = e
                sacc = sacc + e
            sv = jnp.zeros((16,), jnp.float32) + jnp.sum(sacc)
            inv = jnp.ones((16,), jnp.float32) / sv

            def o_c(c, accs):
                e_c = ebuf[pl.ds(c * 16, 16)]
                new = list(accs)
                for j in range(16):
                    eb = _bcast_lane(e_c, j)
                    lrow = c * 16 + j
                    for k in range(4):
                        g = gbuf[slot, h + 1, lrow, pl.ds(16 * k, 16)]
                        a = k * 2 + (j % 2)
                        new[a] = new[a] + eb * g
                return tuple(new)

            accs = lax.fori_loop(0, NCH, o_c, (zero,) * 8)
            for k in range(4):
                ok = accs[k * 2] + accs[k * 2 + 1]
                ubuf[pl.ds(16 * k, 16)] = up[k] + inv * ok
            return 0

        lax.fori_loop(0, HOPS, hop_body, 0)
        pltpu.sync_copy(ubuf, u_hbm.at[row - ci * CB])

    fire(row0, 0)

    def pair_body(i, _):
        r = row0 + 2 * i
        for s in (0, 1):
            row = r + s
            nxt = row + 1

            @pl.when(nxt < row0 + RPW)
            def _():
                fire(nxt, 1 - s)

            drain(row, s)
            compute(row, s)
        return 0

    lax.fori_loop(0, RPW // 2, pair_body, 0)


def _sc_attention(ci, ids, hidden, t0, t1, t2, t3):
    mesh = plsc.VectorSubcoreMesh(core_axis_name="c", subcore_axis_name="s")
    return pl.kernel(
        functools.partial(_sc_attn_body, ci),
        out_type=jax.ShapeDtypeStruct((CB, D), jnp.float32),
        mesh=mesh,
        scratch_types=[
            pltpu.VMEM((2, NT, LP, D), jnp.float32),
            pltpu.VMEM((2, LP), jnp.int32),
            pltpu.VMEM((D,), jnp.float32),
            pltpu.VMEM((LP,), jnp.float32),
            pltpu.VMEM((LP,), jnp.float32),
            pltpu.SemaphoreType.DMA,
            pltpu.SemaphoreType.DMA,
        ],
        compiler_params=pltpu.CompilerParams(
            use_tc_tiling_on_sc=False, needs_layout_passes=False),
    )(ids, hidden, t0, t1, t2, t3)


VB = 512


def _classifier(us, Wt, bt):
    nvb = pl.cdiv(V, VB)

    def body(u0, u1, u2, u3, w_ref, b_ref, o_ref):
        uref = [u0, u1, u2, u3]
        for k in range(NB):
            o_ref[:, pl.ds(k * CB, CB)] = jax.nn.sigmoid(
                lax.dot_general(
                    w_ref[...], uref[k][...],
                    dimension_numbers=(((0,), (1,)), ((), ())),
                    preferred_element_type=jnp.float32,
                ) + b_ref[...])

    uspec = pl.BlockSpec((CB, D), lambda j: (0, 0))
    return pl.pallas_call(
        body,
        grid=(nvb,),
        in_specs=[uspec] * NB + [
            pl.BlockSpec((D, VB), lambda j: (0, j)),
            pl.BlockSpec((VB, 1), lambda j: (j, 0)),
        ],
        out_specs=pl.BlockSpec((VB, B), lambda j: (j, 0)),
        out_shape=jax.ShapeDtypeStruct((V, B), jnp.float32),
    )(*us, Wt, bt)


def kernel(input_ids, hidden_states, C0, C1, C2, C3, W, b):
    ids = input_ids.astype(jnp.int32)
    us = [_sc_attention(ci, ids, hidden_states, C0, C1, C2, C3)
          for ci in range(NB)]
    out_t = _classifier(us, W.T, b.reshape(V, 1))
    return out_t.T
